# Initial kernel scaffold; baseline (speedup 1.0000x reference)
import jax
import jax.numpy as jnp
from jax.experimental import pallas as pl


def kernel(x, edge_index, batch, params):
    raise NotImplementedError("write your pallas kernel here")


# speedup vs baseline: 14.3217x; 14.3217x over previous
import functools

import jax
import jax.numpy as jnp
from jax import lax
from jax.experimental import pallas as pl
from jax.experimental.pallas import tpu as pltpu
from jax.experimental.pallas import tpu_sc as plsc

N_NODES = 10000
N_EDGES = 320000
D_FEAT = 128
HID = 64
N_GRAPHS = 64
N_LAYERS = 3

NC = 2
NS = 16
NW = NC * NS
EPW = N_EDGES // NW
CH = 80
NCH = EPW // CH
NBUF = 5
RPS = 624
TAIL0 = NS * RPS
TAILN = N_NODES - TAIL0

_f32 = jnp.float32


def _segsum_body(h_hbm, src_hbm, dst_hbm, zero_hbm, out_hbm,
                 src_v, dst_v, rows_v, acc, sems):
    c = lax.axis_index("c")
    s = lax.axis_index("s")
    wid = s * NC + c

    pltpu.sync_copy(src_hbm.at[wid], src_v)
    pltpu.sync_copy(dst_hbm.at[wid], dst_v)

    r0 = s * RPS
    pltpu.sync_copy(zero_hbm.at[pl.ds(r0, RPS)], acc.at[pl.ds(r0, RPS)])

    @pl.when(s == NS - 1)
    def _():
        pltpu.sync_copy(zero_hbm.at[pl.ds(TAIL0, TAILN)],
                        acc.at[pl.ds(TAIL0, TAILN)])

    plsc.subcore_barrier()

    for b in range(NBUF):
        pltpu.async_copy(h_hbm.at[src_v.at[b]], rows_v.at[b], sems.at[b])

    n_outer = NCH // NBUF

    def outer(g, carry):
        for b in range(NBUF):
            j = g * NBUF + b
            pltpu.make_async_copy(
                h_hbm.at[src_v.at[j]], rows_v.at[b], sems.at[b]).wait()
            pltpu.sync_copy(rows_v.at[b], acc.at[dst_v.at[j]], add=True)

            @pl.when(g < n_outer - 1)
            def _():
                pltpu.async_copy(
                    h_hbm.at[src_v.at[j + NBUF]], rows_v.at[b], sems.at[b])
        return carry

    lax.fori_loop(0, n_outer, outer, 0)
    plsc.subcore_barrier()

    pltpu.sync_copy(acc.at[pl.ds(r0, RPS)], out_hbm.at[c, pl.ds(r0, RPS)])

    @pl.when(s == NS - 1)
    def _():
        pltpu.sync_copy(acc.at[pl.ds(TAIL0, TAILN)],
                        out_hbm.at[c, pl.ds(TAIL0, TAILN)])


@functools.cache
def _get_segsum():
    return pl.kernel(
        _segsum_body,
        out_type=jax.ShapeDtypeStruct((NC, N_NODES, HID), _f32),
        mesh=plsc.VectorSubcoreMesh(core_axis_name="c", subcore_axis_name="s"),
        scratch_types=[
            pltpu.VMEM((NCH, CH), jnp.int32),
            pltpu.VMEM((NCH, CH), jnp.int32),
            pltpu.VMEM((NBUF, CH, HID), _f32),
            pltpu.VMEM_SHARED((N_NODES, HID), _f32),
            pltpu.SemaphoreType.DMA((NBUF,)),
        ],
        compiler_params=pltpu.CompilerParams(use_tc_tiling_on_sc=False),
    )


def _bn_relu(t, g, be):
    m = jnp.mean(t, axis=0, keepdims=True)
    v = jnp.mean((t - m) ** 2, axis=0, keepdims=True)
    return jnp.maximum(g * (t - m) / jnp.sqrt(v + 1e-5) + be, 0.0)


def _mlp(y, w1, b1, g1, be1, w2, b2, g2, be2):
    t = jnp.dot(y, w1, preferred_element_type=_f32) + b1
    t = _bn_relu(t, g1, be1)
    u = jnp.dot(t, w2, preferred_element_type=_f32) + b2
    return _bn_relu(u, g2, be2)


def _first_body(x_ref, w1, b1, g1, be1, w2, b2, g2, be2, wl,
                h_ref, z_ref):
    h = _mlp(x_ref[...], w1[...], b1[...], g1[...], be1[...],
             w2[...], b2[...], g2[...], be2[...])
    h_ref[...] = h
    z_ref[...] = jnp.sum(h * wl[...], axis=1, keepdims=True)


_first = pl.pallas_call(
    _first_body,
    out_shape=(jax.ShapeDtypeStruct((N_NODES, HID), _f32),
               jax.ShapeDtypeStruct((N_NODES, 1), _f32)),
)


def _layer_body(h_ref, p_ref, w1, b1, g1, be1, w2, b2, g2, be2, wl,
                h_ref_out, z_ref):
    y = h_ref[...] + p_ref[0] + p_ref[1]
    h = _mlp(y, w1[...], b1[...], g1[...], be1[...],
             w2[...], b2[...], g2[...], be2[...])
    h_ref_out[...] = h
    z_ref[...] = jnp.sum(h * wl[...], axis=1, keepdims=True)


_layer = pl.pallas_call(
    _layer_body,
    out_shape=(jax.ShapeDtypeStruct((N_NODES, HID), _f32),
               jax.ShapeDtypeStruct((N_NODES, 1), _f32)),
)


def _pool_body(z0, z1, z2, z3, batch_ref, b0, b1, b2, b3, out_ref):
    z = z0[...] + z1[...] + z2[...] + z3[...]
    gids = lax.broadcasted_iota(jnp.int32, (N_NODES, N_GRAPHS), 1)
    onehot = (batch_ref[...] == gids).astype(_f32)
    s = jnp.sum(z * onehot, axis=0, keepdims=True)
    cnt = jnp.sum(onehot, axis=0, keepdims=True)
    bsum = b0[0, 0] + b1[0, 0] + b2[0, 0] + b3[0, 0]
    out_ref[...] = jax.nn.sigmoid((s + bsum * cnt) / jnp.maximum(cnt, 1.0))


_pool = pl.pallas_call(
    _pool_body,
    out_shape=jax.ShapeDtypeStruct((1, N_GRAPHS), _f32),
)


def kernel(x, edge_index, batch, params):
    src3 = edge_index[0].reshape(NW, NCH, CH)
    dst3 = edge_index[1].reshape(NW, NCH, CH)
    zeros = jnp.zeros((N_NODES, HID), _f32)

    def row(v):
        return v.reshape(1, HID)

    fp = params['first_h']
    h, z0 = _first(x, fp['W1'], row(fp['b1']), row(fp['g1']), row(fp['be1']),
                   fp['W2'], row(fp['b2']), row(fp['g2']), row(fp['be2']),
                   params['lin_W'][0].reshape(1, HID))
    zs = [z0]
    segsum = _get_segsum()
    for l in range(N_LAYERS):
        parts = segsum(h, src3, dst3, zeros)
        np_ = params['nns'][l]
        h, zl = _layer(h, parts,
                       np_['W1'], row(np_['b1']), row(np_['g1']), row(np_['be1']),
                       np_['W2'], row(np_['b2']), row(np_['g2']), row(np_['be2']),
                       params['lin_W'][l + 1].reshape(1, HID))
        zs.append(zl)

    out = _pool(zs[0], zs[1], zs[2], zs[3], batch.reshape(N_NODES, 1),
                params['lin_b'][0].reshape(1, 1),
                params['lin_b'][1].reshape(1, 1),
                params['lin_b'][2].reshape(1, 1),
                params['lin_b'][3].reshape(1, 1))
    return out.reshape(-1)
